# Initial kernel scaffold; baseline (speedup 1.0000x reference)
import jax
import jax.numpy as jnp
from jax.experimental import pallas as pl


def kernel(x, edge_index, W, b, att_i, att_j):
    raise NotImplementedError("write your pallas kernel here")


# speedup vs baseline: 3.4630x; 3.4630x over previous
import functools

import jax
import jax.numpy as jnp
from jax.experimental import pallas as pl

MIN = 1e-15
MAXN = 0.996

N = 10000
D = 128
OC = 64
ROWS = 1000


def _artanh(x):
    x = jnp.clip(x, -1.0 + 1e-7, 1.0 - 1e-7)
    return 0.5 * jnp.log((1.0 + x) / (1.0 - x))


def _nrm(x):
    return jnp.maximum(
        jnp.sqrt(jnp.sum(x * x, axis=-1, keepdims=True)), MIN)


def _proj(x):
    n = _nrm(x)
    return jnp.where(n > MAXN, x / n * MAXN, x)


def _stage_a_body(x_ref, wt_ref, b_ref, ai_ref, aj_ref, xt_ref, s_ref):
    xv = x_ref[...]
    mx = jnp.dot(xv, wt_ref[...], preferred_element_type=jnp.float32)
    xn = _nrm(xv)
    mxn = _nrm(mx)
    res = jnp.tanh(mxn / xn * _artanh(xn)) * mx / mxn
    cond = jnp.all(mx == 0.0, axis=-1, keepdims=True)
    res = jnp.where(cond, 0.0, res)
    res = _proj(res)
    bv = b_ref[...]
    bn = _nrm(bv)
    hb = _proj(jnp.tanh(bn) * bv / bn)
    x2 = jnp.sum(res * res, axis=-1, keepdims=True)
    y2 = jnp.sum(hb * hb, axis=-1, keepdims=True)
    xy = jnp.sum(res * hb, axis=-1, keepdims=True)
    num = (1.0 + 2.0 * xy + y2) * res + (1.0 - x2) * hb
    den = jnp.maximum(1.0 + 2.0 * xy + x2 * y2, MIN)
    h = _proj(num / den)
    hn = _nrm(h)
    xt = _artanh(hn) * h / hn
    xt_ref[...] = xt
    ai = ai_ref[...]
    aj = aj_ref[...]
    si0 = jnp.sum(xt[:, :OC] * ai[:, :OC], axis=-1)
    si1 = jnp.sum(xt[:, OC:] * ai[:, OC:], axis=-1)
    sj0 = jnp.sum(xt[:, :OC] * aj[:, :OC], axis=-1)
    sj1 = jnp.sum(xt[:, OC:] * aj[:, OC:], axis=-1)
    z = jnp.zeros_like(si0)
    s_ref[...] = jnp.stack([si0, si1, sj0, sj1, z, z, z, z], axis=-1)


def _stage_a(x, wt, b2, ai, aj):
    return pl.pallas_call(
        _stage_a_body,
        grid=(N // ROWS,),
        in_specs=[
            pl.BlockSpec((ROWS, D), lambda i: (i, 0)),
            pl.BlockSpec((D, D), lambda i: (0, 0)),
            pl.BlockSpec((1, D), lambda i: (0, 0)),
            pl.BlockSpec((1, D), lambda i: (0, 0)),
            pl.BlockSpec((1, D), lambda i: (0, 0)),
        ],
        out_specs=[
            pl.BlockSpec((ROWS, D), lambda i: (i, 0)),
            pl.BlockSpec((ROWS, 8), lambda i: (i, 0)),
        ],
        out_shape=[
            jax.ShapeDtypeStruct((N, D), jnp.float32),
            jax.ShapeDtypeStruct((N, 8), jnp.float32),
        ],
    )(x, wt, b2, ai, aj)


def _stage_c_body(p_ref, xt_ref, s_ref, out_ref):
    p = p_ref[0] + p_ref[1]
    xtv = xt_ref[...]
    aL0 = s_ref[:, 0:1] + s_ref[:, 2:3]
    aL1 = s_ref[:, 1:2] + s_ref[:, 3:4]
    cL0 = jnp.exp(jnp.maximum(aL0, 0.2 * aL0))
    cL1 = jnp.exp(jnp.maximum(aL1, 0.2 * aL1))
    den0 = p[:, 128:129] + cL0
    den1 = p[:, 129:130] + cL1
    sup0 = p[:, 0:OC] + cL0 * xtv[:, :OC]
    sup1 = p[:, OC:D] + cL1 * xtv[:, OC:]
    support = 0.5 * (sup0 / jnp.maximum(den0, 1e-16)
                     + sup1 / jnp.maximum(den1, 1e-16))
    n = _nrm(support)
    h = _proj(jnp.tanh(n) * support / n)
    hn = _nrm(h)
    xt2 = _artanh(hn) * h / hn
    xt2 = jnp.where(xt2 > 0, xt2, 0.01 * xt2)
    n2 = _nrm(xt2)
    out = _proj(jnp.tanh(n2) * xt2 / n2)
    out_ref[...] = out


def _stage_c(p, xt, s):
    return pl.pallas_call(
        _stage_c_body,
        grid=(N // ROWS,),
        in_specs=[
            pl.BlockSpec((2, ROWS, 144), lambda i: (0, i, 0)),
            pl.BlockSpec((ROWS, D), lambda i: (i, 0)),
            pl.BlockSpec((ROWS, 8), lambda i: (i, 0)),
        ],
        out_specs=pl.BlockSpec((ROWS, OC), lambda i: (i, 0)),
        out_shape=jax.ShapeDtypeStruct((N, OC), jnp.float32),
    )(p, xt, s)


def kernel(x, edge_index, W, b, att_i, att_j):
    wt = W.T
    b2 = b.reshape(1, D)
    ai = att_i.reshape(1, D)
    aj = att_j.reshape(1, D)
    xt, s = _stage_a(x, wt, b2, ai, aj)

    src, dst = edge_index[0], edge_index[1]
    si0, si1, sj0, sj1 = s[:, 0], s[:, 1], s[:, 2], s[:, 3]
    a0 = si0[src] + sj0[dst]
    a1 = si1[src] + sj1[dst]
    a0 = jnp.maximum(a0, 0.2 * a0)
    a1 = jnp.maximum(a1, 0.2 * a1)
    valid = src != dst
    c0 = jnp.where(valid, jnp.exp(a0), 0.0)
    c1 = jnp.where(valid, jnp.exp(a1), 0.0)
    xj = xt[dst]
    upd = jnp.concatenate([
        c0[:, None] * xj[:, :OC],
        c1[:, None] * xj[:, OC:],
        c0[:, None], c1[:, None],
        jnp.zeros((src.shape[0], 14), jnp.float32),
    ], axis=-1)
    p0 = jax.ops.segment_sum(upd, src, num_segments=N)
    p = jnp.stack([p0, jnp.zeros_like(p0)])

    return _stage_c(p, xt, s)


# speedup vs baseline: 45.0054x; 12.9959x over previous
import functools

import jax
import jax.numpy as jnp
from jax import lax
from jax.experimental import pallas as pl
from jax.experimental.pallas import tpu as pltpu
from jax.experimental.pallas import tpu_sc as plsc

MIN = 1e-15
MAXN = 0.996

N = 10000
D = 128
OC = 64
ROWS = 1000

NCORE = 2
NSUB = 16
CHUNK = 128
NCHUNK = 158
EW = NCHUNK * CHUNK
EPAD = EW * NSUB
WIDTH = 80
NPAD = 10240
RPT = NPAD // NSUB


def _artanh(x):
    x = jnp.clip(x, -1.0 + 1e-7, 1.0 - 1e-7)
    return 0.5 * jnp.log((1.0 + x) / (1.0 - x))


def _nrm(x):
    return jnp.maximum(
        jnp.sqrt(jnp.sum(x * x, axis=-1, keepdims=True)), MIN)


def _proj(x):
    n = _nrm(x)
    return jnp.where(n > MAXN, x / n * MAXN, x)


def _stage_a_body(x_ref, wt_ref, b_ref, ai_ref, aj_ref, xth_ref, s_ref):
    xv = x_ref[...]
    mx = jnp.dot(xv, wt_ref[...], preferred_element_type=jnp.float32)
    xn = _nrm(xv)
    mxn = _nrm(mx)
    res = jnp.tanh(mxn / xn * _artanh(xn)) * mx / mxn
    cond = jnp.all(mx == 0.0, axis=-1, keepdims=True)
    res = jnp.where(cond, 0.0, res)
    res = _proj(res)
    bv = b_ref[...]
    bn = _nrm(bv)
    hb = _proj(jnp.tanh(bn) * bv / bn)
    x2 = jnp.sum(res * res, axis=-1, keepdims=True)
    y2 = jnp.sum(hb * hb, axis=-1, keepdims=True)
    xy = jnp.sum(res * hb, axis=-1, keepdims=True)
    num = (1.0 + 2.0 * xy + y2) * res + (1.0 - x2) * hb
    den = jnp.maximum(1.0 + 2.0 * xy + x2 * y2, MIN)
    h = _proj(num / den)
    hn = _nrm(h)
    xt = _artanh(hn) * h / hn
    xth_ref[0] = xt[:, :OC]
    xth_ref[1] = xt[:, OC:]
    ai = ai_ref[...]
    aj = aj_ref[...]
    si0 = jnp.sum(xt[:, :OC] * ai[:, :OC], axis=-1)
    si1 = jnp.sum(xt[:, OC:] * ai[:, OC:], axis=-1)
    sj0 = jnp.sum(xt[:, :OC] * aj[:, :OC], axis=-1)
    sj1 = jnp.sum(xt[:, OC:] * aj[:, OC:], axis=-1)
    z = jnp.zeros_like(si0)
    s_ref[...] = jnp.stack([si0, si1, sj0, sj1, z, z, z, z], axis=-1)


def _stage_a(x, wt, b2, ai, aj):
    return pl.pallas_call(
        _stage_a_body,
        grid=(N // ROWS,),
        in_specs=[
            pl.BlockSpec((ROWS, D), lambda i: (i, 0)),
            pl.BlockSpec((D, D), lambda i: (0, 0)),
            pl.BlockSpec((1, D), lambda i: (0, 0)),
            pl.BlockSpec((1, D), lambda i: (0, 0)),
            pl.BlockSpec((1, D), lambda i: (0, 0)),
        ],
        out_specs=[
            pl.BlockSpec((2, ROWS, OC), lambda i: (0, i, 0)),
            pl.BlockSpec((ROWS, 8), lambda i: (i, 0)),
        ],
        out_shape=[
            jax.ShapeDtypeStruct((2, N, OC), jnp.float32),
            jax.ShapeDtypeStruct((N, 8), jnp.float32),
        ],
    )(x, wt, b2, ai, aj)


def _stage_c_body(p_ref, xth_ref, s_ref, out_ref):
    p0 = p_ref[0]
    p1 = p_ref[1]
    xt0 = xth_ref[0]
    xt1 = xth_ref[1]
    aL0 = s_ref[:, 0:1] + s_ref[:, 2:3]
    aL1 = s_ref[:, 1:2] + s_ref[:, 3:4]
    cL0 = jnp.exp(jnp.maximum(aL0, 0.2 * aL0))
    cL1 = jnp.exp(jnp.maximum(aL1, 0.2 * aL1))
    den0 = p0[:, OC:OC + 1] + cL0
    den1 = p1[:, OC:OC + 1] + cL1
    sup0 = p0[:, 0:OC] + cL0 * xt0
    sup1 = p1[:, 0:OC] + cL1 * xt1
    support = 0.5 * (sup0 / jnp.maximum(den0, 1e-16)
                     + sup1 / jnp.maximum(den1, 1e-16))
    n = _nrm(support)
    h = _proj(jnp.tanh(n) * support / n)
    hn = _nrm(h)
    xt2 = _artanh(hn) * h / hn
    xt2 = jnp.where(xt2 > 0, xt2, 0.01 * xt2)
    n2 = _nrm(xt2)
    out = _proj(jnp.tanh(n2) * xt2 / n2)
    out_ref[...] = out


def _stage_c(p, xth, s):
    return pl.pallas_call(
        _stage_c_body,
        grid=(N // ROWS,),
        in_specs=[
            pl.BlockSpec((2, ROWS, WIDTH), lambda i: (0, i, 0)),
            pl.BlockSpec((2, ROWS, OC), lambda i: (0, i, 0)),
            pl.BlockSpec((ROWS, 8), lambda i: (i, 0)),
        ],
        out_specs=pl.BlockSpec((ROWS, OC), lambda i: (i, 0)),
        out_shape=jax.ShapeDtypeStruct((N, OC), jnp.float32),
    )(p, xth, s)


def _edge_body(src_hbm, dst_hbm, sa_hbm, sb_hbm, xtc_hbm,
               out_hbm, sa_v, sb_v, srcv, dstv, rows, outr, acc, sem):
    cid = lax.axis_index("c")
    sid = lax.axis_index("s")
    cidN = cid * N

    pltpu.sync_copy(sa_hbm, sa_v)
    pltpu.sync_copy(sb_hbm, sb_v)

    zeros16 = jnp.zeros((16,), jnp.float32)

    def zero_row(r, _):
        for j in range(WIDTH // 16):
            outr[r, pl.ds(j * 16, 16)] = zeros16
        return 0

    lax.fori_loop(0, CHUNK, zero_row, 0)
    stripe0 = sid * RPT
    for k in range(RPT // CHUNK):
        pltpu.sync_copy(outr, acc.at[pl.ds(stripe0 + k * CHUNK, CHUNK)])
    plsc.subcore_barrier()

    ebase = sid * EW
    col_c = jnp.full((16,), OC, jnp.int32)

    def chunk_body(t, _):
        b = ebase + t * CHUNK
        pltpu.sync_copy(src_hbm.at[pl.ds(b, CHUNK)], srcv)
        pltpu.sync_copy(dst_hbm.at[pl.ds(cid * EPAD + b, CHUNK)], dstv)
        pltpu.async_copy(xtc_hbm.at[dstv], rows, sem).wait()

        def group_body(g, _):
            r0 = g * 16
            sv = srcv[pl.ds(r0, 16)]
            dv = dstv[pl.ds(r0, 16)]
            sva = sv + cidN
            s0 = plsc.load_gather(sa_v, [sva])
            t0 = plsc.load_gather(sb_v, [dv])
            a = s0 + t0
            c = jnp.exp(jnp.maximum(a, 0.2 * a))
            c = jnp.where(sva != dv, c, zeros16)
            rid = r0 + lax.broadcasted_iota(jnp.int32, (16,), 0)
            plsc.store_scatter(outr, [rid, col_c], c)
            for e in range(16):
                ce = c[e]
                r = r0 + e
                for j in range(OC // 16):
                    outr[r, pl.ds(j * 16, 16)] = (
                        rows[r, pl.ds(j * 16, 16)] * ce)
            return 0

        lax.fori_loop(0, CHUNK // 16, group_body, 0)
        pltpu.sync_copy(outr, acc.at[srcv], add=True)
        return 0

    lax.fori_loop(0, NCHUNK, chunk_body, 0)
    plsc.subcore_barrier()

    for k in range(RPT // CHUNK):
        roff = stripe0 + k * CHUNK
        pltpu.sync_copy(acc.at[pl.ds(roff, CHUNK)],
                        out_hbm.at[pl.ds(cid * NPAD + roff, CHUNK)])


_edge_kernel = functools.partial(
    pl.kernel,
    out_type=jax.ShapeDtypeStruct((NCORE * NPAD, WIDTH), jnp.float32),
    mesh=plsc.VectorSubcoreMesh(core_axis_name="c", subcore_axis_name="s"),
    compiler_params=pltpu.CompilerParams(
        needs_layout_passes=False, use_tc_tiling_on_sc=False),
    scratch_types=[
        pltpu.VMEM((NCORE * N,), jnp.float32),
        pltpu.VMEM((NCORE * N,), jnp.float32),
        pltpu.VMEM((CHUNK,), jnp.int32),
        pltpu.VMEM((CHUNK,), jnp.int32),
        pltpu.VMEM((CHUNK, OC), jnp.float32),
        pltpu.VMEM((CHUNK, WIDTH), jnp.float32),
        pltpu.VMEM_SHARED((NPAD, WIDTH), jnp.float32),
        pltpu.SemaphoreType.DMA,
    ],
)(_edge_body)


def kernel(x, edge_index, W, b, att_i, att_j):
    wt = W.T
    b2 = b.reshape(1, D)
    ai = att_i.reshape(1, D)
    aj = att_j.reshape(1, D)
    xth, s = _stage_a(x, wt, b2, ai, aj)

    src, dst = edge_index[0], edge_index[1]
    npad = EPAD - src.shape[0]
    pad = (jnp.arange(npad, dtype=src.dtype) * 101) % N
    srcp = jnp.concatenate([src, pad])
    dstp = jnp.concatenate([dst, pad])
    dst2 = jnp.concatenate([dstp, dstp + N])
    sa = jnp.concatenate([s[:, 0], s[:, 1]])
    sb = jnp.concatenate([s[:, 2], s[:, 3]])
    xtc = xth.reshape(NCORE * N, OC)
    p = _edge_kernel(srcp, dst2, sa, sb, xtc)
    p = p.reshape(NCORE, NPAD, WIDTH)

    return _stage_c(p, xth, s)


# speedup vs baseline: 158.2113x; 3.5154x over previous
import functools

import jax
import jax.numpy as jnp
from jax import lax
from jax.experimental import pallas as pl
from jax.experimental.pallas import tpu as pltpu
from jax.experimental.pallas import tpu_sc as plsc

MIN = 1e-15
MAXN = 0.996

N = 10000
D = 128
OC = 64
ROWS = 1000

NCORE = 2
NSUB = 16
CHUNK = 128
NCHUNK = 160
EW = NCHUNK * CHUNK
EPAD = EW * NSUB
E_TOTAL = 320000
WIDTH = 80
NPAD = N
RPT = NPAD // NSUB
RCP = 125


def _artanh(x):
    x = jnp.clip(x, -1.0 + 1e-7, 1.0 - 1e-7)
    return 0.5 * jnp.log((1.0 + x) / (1.0 - x))


def _nrm(x):
    return jnp.maximum(
        jnp.sqrt(jnp.sum(x * x, axis=-1, keepdims=True)), MIN)


def _proj(x):
    n = _nrm(x)
    return jnp.where(n > MAXN, x / n * MAXN, x)


EBLK = EPAD // 10


def _stage_a_body(x_ref, w_ref, b_ref, ai_ref, aj_ref, ei_ref,
                  xt_ref, si_ref, sj_ref, sp_ref, dp_ref):
    i = pl.program_id(0)
    eid = (i * EBLK + lax.broadcasted_iota(jnp.int32, (1, EBLK), 1))
    pad_val = eid % N
    oob = eid >= E_TOTAL
    sp = jnp.where(oob, pad_val, ei_ref[0:1, :])
    dp = jnp.where(oob, pad_val, ei_ref[1:2, :])
    sp_ref[...] = sp.reshape(EBLK // CHUNK, CHUNK)
    dp_ref[...] = dp.reshape(EBLK // CHUNK, CHUNK)
    xv = x_ref[...]
    mx = lax.dot_general(xv, w_ref[...], (((1,), (1,)), ((), ())),
                         preferred_element_type=jnp.float32)
    xn = _nrm(xv)
    mxn = _nrm(mx)
    res = jnp.tanh(mxn / xn * _artanh(xn)) * mx / mxn
    cond = jnp.all(mx == 0.0, axis=-1, keepdims=True)
    res = jnp.where(cond, 0.0, res)
    res = _proj(res)
    bv = b_ref[...]
    bn = _nrm(bv)
    hb = _proj(jnp.tanh(bn) * bv / bn)
    x2 = jnp.sum(res * res, axis=-1, keepdims=True)
    y2 = jnp.sum(hb * hb, axis=-1, keepdims=True)
    xy = jnp.sum(res * hb, axis=-1, keepdims=True)
    num = (1.0 + 2.0 * xy + y2) * res + (1.0 - x2) * hb
    den = jnp.maximum(1.0 + 2.0 * xy + x2 * y2, MIN)
    h = _proj(num / den)
    hn = _nrm(h)
    xt = _artanh(hn) * h / hn
    xt_ref[...] = xt
    ai = ai_ref[...]
    aj = aj_ref[...]
    si0 = jnp.sum(xt[:, :OC] * ai[:, :OC], axis=-1)
    si1 = jnp.sum(xt[:, OC:] * ai[:, OC:], axis=-1)
    sj0 = jnp.sum(xt[:, :OC] * aj[:, :OC], axis=-1)
    sj1 = jnp.sum(xt[:, OC:] * aj[:, OC:], axis=-1)
    si_ref[...] = jnp.stack([si0, si1], axis=-1)
    sj_ref[...] = jnp.stack([sj0, sj1], axis=-1)


def _stage_a(x, w, b2, ai, aj, edge_index):
    return pl.pallas_call(
        _stage_a_body,
        grid=(N // ROWS,),
        in_specs=[
            pl.BlockSpec((ROWS, D), lambda i: (i, 0)),
            pl.BlockSpec((D, D), lambda i: (0, 0)),
            pl.BlockSpec((1, D), lambda i: (0, 0)),
            pl.BlockSpec((1, D), lambda i: (0, 0)),
            pl.BlockSpec((1, D), lambda i: (0, 0)),
            pl.BlockSpec((2, EBLK), lambda i: (0, i)),
        ],
        out_specs=[
            pl.BlockSpec((ROWS, D), lambda i: (i, 0)),
            pl.BlockSpec((ROWS, 2), lambda i: (i, 0)),
            pl.BlockSpec((ROWS, 2), lambda i: (i, 0)),
            pl.BlockSpec((EBLK // CHUNK, CHUNK), lambda i: (i, 0)),
            pl.BlockSpec((EBLK // CHUNK, CHUNK), lambda i: (i, 0)),
        ],
        out_shape=[
            jax.ShapeDtypeStruct((N, D), jnp.float32),
            jax.ShapeDtypeStruct((N, 2), jnp.float32),
            jax.ShapeDtypeStruct((N, 2), jnp.float32),
            jax.ShapeDtypeStruct((NSUB * NCHUNK, CHUNK), jnp.int32),
            jax.ShapeDtypeStruct((NSUB * NCHUNK, CHUNK), jnp.int32),
        ],
    )(x, w, b2, ai, aj, edge_index)


def _stage_c_body(p0_ref, p1_ref, xt_ref, si_ref, sj_ref, out_ref):
    p0 = p0_ref[...]
    p1 = p1_ref[...]
    xt0 = xt_ref[:, :OC]
    xt1 = xt_ref[:, OC:]
    aL0 = si_ref[:, 0:1] + sj_ref[:, 0:1]
    aL1 = si_ref[:, 1:2] + sj_ref[:, 1:2]
    cL0 = jnp.exp(jnp.maximum(aL0, 0.2 * aL0))
    cL1 = jnp.exp(jnp.maximum(aL1, 0.2 * aL1))
    den0 = p0[:, OC:OC + 1] + cL0
    den1 = p1[:, OC:OC + 1] + cL1
    sup0 = p0[:, 0:OC] + cL0 * xt0
    sup1 = p1[:, 0:OC] + cL1 * xt1
    support = 0.5 * (sup0 / jnp.maximum(den0, 1e-16)
                     + sup1 / jnp.maximum(den1, 1e-16))
    n = _nrm(support)
    h = _proj(jnp.tanh(n) * support / n)
    hn = _nrm(h)
    xt2 = _artanh(hn) * h / hn
    xt2 = jnp.where(xt2 > 0, xt2, 0.01 * xt2)
    n2 = _nrm(xt2)
    out = _proj(jnp.tanh(n2) * xt2 / n2)
    out_ref[...] = out


def _stage_c(p, xt, si2, sj2):
    return pl.pallas_call(
        _stage_c_body,
        grid=(N // ROWS,),
        in_specs=[
            pl.BlockSpec((ROWS, WIDTH), lambda i: (i, 0)),
            pl.BlockSpec((ROWS, WIDTH), lambda i: (N // ROWS + i, 0)),
            pl.BlockSpec((ROWS, D), lambda i: (i, 0)),
            pl.BlockSpec((ROWS, 2), lambda i: (i, 0)),
            pl.BlockSpec((ROWS, 2), lambda i: (i, 0)),
        ],
        out_specs=pl.BlockSpec((ROWS, OC), lambda i: (i, 0)),
        out_shape=jax.ShapeDtypeStruct((N, OC), jnp.float32),
    )(p, p, xt, si2, sj2)


NSLOT = 16


def _edge_body(src_hbm, dst_hbm, si_hbm, sj_hbm, xtc_hbm, out_hbm,
               si_v, sj_v, sbuf, dbuf, rows0, rows1, rows2, rows3,
               outr0, outr1, acc,
               g0, g1, g2, g3, sf0, sf1, diA, diB):
    cid = lax.axis_index("c")
    sid = lax.axis_index("s")
    srow0 = sid * NCHUNK
    rows = [rows0, rows1, rows2, rows3]
    gsem = [g0, g1, g2, g3]
    outr = [outr0, outr1]
    sfsem = [sf0, sf1]
    disem = [diA, diB]

    pltpu.sync_copy(si_hbm.at[pl.ds(cid * N, N)], si_v)
    pltpu.sync_copy(sj_hbm.at[pl.ds(cid * N, N)], sj_v)

    zeros16 = jnp.zeros((16,), jnp.float32)

    def zero_row(r, _):
        for j in range(WIDTH // 16):
            outr0[r, pl.ds(j * 16, 16)] = zeros16
            outr1[r, pl.ds(j * 16, 16)] = zeros16
        return 0

    lax.fori_loop(0, CHUNK, zero_row, 0)
    stripe0 = sid * RPT
    for k in range(RPT // RCP):
        pltpu.sync_copy(outr0.at[pl.ds(0, RCP)],
                        acc.at[pl.ds(stripe0 + k * RCP, RCP)])
    plsc.subcore_barrier()

    col_c = jnp.full((16,), OC, jnp.int32)
    iota16 = lax.broadcasted_iota(jnp.int32, (16,), 0)

    def idx_batch_load(b, sem):
        for k in range(4):
            t = 4 * b + k
            tc = jnp.minimum(t, NCHUNK - 1)
            slot = lax.rem(t, NSLOT)
            pltpu.async_copy(src_hbm.at[srow0 + tc], sbuf.at[slot], sem)
            pltpu.async_copy(dst_hbm.at[srow0 + tc], dbuf.at[slot], sem)

    def idx_batch_wait(b, sem):
        for _ in range(8):
            pltpu.make_async_copy(src_hbm.at[srow0], sbuf.at[0], sem).wait()
        for k in range(4):
            slot = lax.rem(4 * b + k, NSLOT)
            for j in range(CHUNK // 16):
                dbuf[slot, pl.ds(j * 16, 16)] = (
                    (dbuf[slot, pl.ds(j * 16, 16)] << 1) + cid)

    def gather(t, rowbuf, sem):
        pltpu.async_copy(xtc_hbm.at[dbuf.at[lax.rem(t, NSLOT)]], rowbuf, sem)

    def gather_wait(rowbuf, sem):
        pltpu.make_async_copy(xtc_hbm.at[dbuf.at[0]], rowbuf, sem).wait()

    def scatter(t, outbuf, sem):
        pltpu.async_copy(
            outbuf, acc.at[sbuf.at[lax.rem(t, NSLOT)]], sem, add=True)

    def scatter_wait(outbuf, sem):
        pltpu.make_async_copy(outbuf, acc.at[sbuf.at[0]], sem).wait()

    def compute_chunk(t, rowbuf, outbuf):
        slot = lax.rem(t, NSLOT)

        def group_body(g, _):
            r0 = g * 16
            sv = sbuf[slot, pl.ds(r0, 16)]
            dv2 = dbuf[slot, pl.ds(r0, 16)]
            dvn = dv2 >> 1
            sg = plsc.load_gather(si_v, [sv])
            tg = plsc.load_gather(sj_v, [dvn])
            a = sg + tg
            c = jnp.exp(jnp.maximum(a, 0.2 * a))
            c = jnp.where(sv != dvn, c, zeros16)
            plsc.store_scatter(outbuf, [r0 + iota16, col_c], c)
            for e in range(0, 16, 4):
                cs = [c[e + k] for k in range(4)]
                rr = [r0 + e + k for k in range(4)]
                vv = [[rowbuf[rr[k], pl.ds(j * 16, 16)] for j in range(4)]
                      for k in range(4)]
                for k in range(4):
                    for j in range(4):
                        outbuf[rr[k], pl.ds(j * 16, 16)] = vv[k][j] * cs[k]
            return 0

        lax.fori_loop(0, CHUNK // 16, group_body, 0)

    idx_batch_load(0, disem[0])
    idx_batch_wait(0, disem[0])
    idx_batch_load(1, disem[1])
    for k in range(4):
        gather(jnp.int32(k), rows[k], gsem[k])

    NB = NCHUNK // 4

    def quad(b, wait_sem, load_sem, load_ok):
        idx_batch_wait(b + 1, wait_sem)

        @pl.when(load_ok)
        def _():
            idx_batch_load(b + 2, load_sem)

        for k in range(4):
            t = 4 * b + k
            gather_wait(rows[k], gsem[k])

            @pl.when(t >= 2)
            def _():
                scatter_wait(outr[k % 2], sfsem[k % 2])

            compute_chunk(t, rows[k], outr[k % 2])
            scatter(t, outr[k % 2], sfsem[k % 2])
            gather(t + 4, rows[k], gsem[k])

    def pipe_body(j, _):
        b0 = 2 * j
        quad(b0, disem[1], disem[0], b0 + 2 <= NB)
        quad(b0 + 1, disem[0], disem[1], b0 + 3 <= NB)
        return 0

    lax.fori_loop(0, NB // 2, pipe_body, 0)
    for k in range(4):
        gather_wait(rows[k], gsem[k])
    scatter_wait(outr0, sf0)
    scatter_wait(outr1, sf1)
    plsc.subcore_barrier()

    for k in range(RPT // RCP):
        roff = stripe0 + k * RCP
        pltpu.sync_copy(acc.at[pl.ds(roff, RCP)],
                        out_hbm.at[pl.ds(cid * NPAD + roff, RCP)])


_edge_kernel = functools.partial(
    pl.kernel,
    out_type=jax.ShapeDtypeStruct((NCORE * NPAD, WIDTH), jnp.float32),
    mesh=plsc.VectorSubcoreMesh(core_axis_name="c", subcore_axis_name="s"),
    compiler_params=pltpu.CompilerParams(
        needs_layout_passes=False, use_tc_tiling_on_sc=False),
    scratch_types=[
        pltpu.VMEM((N,), jnp.float32),
        pltpu.VMEM((N,), jnp.float32),
        pltpu.VMEM((NSLOT, CHUNK), jnp.int32),
        pltpu.VMEM((NSLOT, CHUNK), jnp.int32),
        pltpu.VMEM((CHUNK, OC), jnp.float32),
        pltpu.VMEM((CHUNK, OC), jnp.float32),
        pltpu.VMEM((CHUNK, OC), jnp.float32),
        pltpu.VMEM((CHUNK, OC), jnp.float32),
        pltpu.VMEM((CHUNK, WIDTH), jnp.float32),
        pltpu.VMEM((CHUNK, WIDTH), jnp.float32),
        pltpu.VMEM_SHARED((NPAD, WIDTH), jnp.float32),
        pltpu.SemaphoreType.DMA,
        pltpu.SemaphoreType.DMA,
        pltpu.SemaphoreType.DMA,
        pltpu.SemaphoreType.DMA,
        pltpu.SemaphoreType.DMA,
        pltpu.SemaphoreType.DMA,
        pltpu.SemaphoreType.DMA,
        pltpu.SemaphoreType.DMA,
    ],
)(_edge_body)


def kernel(x, edge_index, W, b, att_i, att_j):
    b2 = b.reshape(1, D)
    ai = att_i.reshape(1, D)
    aj = att_j.reshape(1, D)
    xt, si2, sj2, srcp, dstp = _stage_a(x, W, b2, ai, aj, edge_index)

    xtc = xt.reshape(NCORE * N, OC)
    sih = jnp.concatenate([si2[:, 0], si2[:, 1]])
    sjh = jnp.concatenate([sj2[:, 0], sj2[:, 1]])
    p = _edge_kernel(srcp, dstp, sih, sjh, xtc)

    return _stage_c(p, xt, si2, sj2)
